# baseline (speedup 1.0000x reference)
import functools

import jax
import jax.numpy as jnp
from jax import lax
from jax.experimental import pallas as pl
from jax.experimental.pallas import tpu as pltpu
from jax.experimental.pallas import tpu_sc as plsc

_L = 16
_NC, _NS = 2, 16
_NW = _NC * _NS


def _dense_kernel(x_ref, h_ref, c_ref, m_ref, packed_ref, mse_ref, *, T, d):
    H = h_ref[...]
    M = m_ref[...]
    dims = (((1,), (1,)), ((), ()))
    G = jax.lax.dot_general(H, H, dims, preferred_element_type=jnp.float32)
    GM = jax.lax.dot_general(M, M, dims, preferred_element_type=jnp.float32)
    nrm = jnp.sum(H * H, axis=1, keepdims=True)
    s = jnp.sum(H, axis=1, keepdims=True)
    mn = jnp.sum(M, axis=1, keepdims=True)

    sqd = nrm + nrm.T - 2.0 * G
    ds = s - s.T
    var = (sqd - ds * ds * (1.0 / d)) * (1.0 / (d - 1.0))
    good = var > 0.0
    score = jnp.where(good, jnp.sqrt(jnp.where(good, var, 1.0)), 0.0)

    msq = mn + mn.T - 2.0 * GM
    iota_r = jax.lax.broadcasted_iota(jnp.int32, (T, T), 0)
    iota_c = jax.lax.broadcasted_iota(jnp.int32, (T, T), 1)
    invalid = (iota_r == iota_c) | (msq <= 0.5)
    packed_ref[:, :T] = jnp.where(invalid, jnp.float32(9999.0), score)

    goodn = sqd > 0.0
    packed_ref[:, T:] = jnp.where(goodn, jnp.sqrt(jnp.where(goodn, sqd, 1.0)), 0.0)

    dd = x_ref[...] - H + c_ref[...]
    mse_ref[...] = jnp.reshape(jnp.sum(M * dd * dd), (1, 1))


def _merge16(ak, av, bk, bv):
    rk = lax.rev(bk, (0,))
    rv = lax.rev(bv, (0,))
    take_a = ak <= rk
    lo_k = jnp.where(take_a, ak, rk)
    lo_v = jnp.where(take_a, av, rv)
    return plsc.sort_key_val(lo_k, lo_v)


def _topk_body(packed_hbm, out_hbm, pk_v, res_v, sem1, sem2, *, T, R):
    wid = lax.axis_index("s") * _NC + lax.axis_index("c")
    base = wid * R
    half = R // 2
    h1 = pltpu.async_copy(packed_hbm.at[pl.ds(base, half)],
                          pk_v.at[pl.ds(0, half)], sem1)
    h2 = pltpu.async_copy(packed_hbm.at[pl.ds(base + half, half)],
                          pk_v.at[pl.ds(half, half)], sem2)
    lane = lax.iota(jnp.int32, _L)
    first8 = lane < 8
    lane0 = lane == 0
    acc = jnp.zeros((_L,), jnp.float32)
    zeros_idx = jnp.zeros((_L,), jnp.int32)
    h1.wait()
    for lo in (0, half):
        if lo:
            h2.wait()
        rows = range(lo, lo + half)
        kept_k = {r: jnp.full((_L,), 3.0e38, jnp.float32) for r in rows}
        kept_v = {r: jnp.zeros((_L,), jnp.float32) for r in rows}
        for c in range(T // _L):
            for r in rows:
                ck = pk_v[r, pl.ds(c * _L, _L)]
                cv = pk_v[r, pl.ds(T + c * _L, _L)]
                sk, sv = plsc.sort_key_val(ck, cv)
                kept_k[r], kept_v[r] = _merge16(kept_k[r], kept_v[r], sk, sv)
        for r in rows:
            v0 = kept_k[r][zeros_idx]
            e = jnp.where(first8, jnp.exp(v0 - kept_k[r]), 0.0)
            num = jnp.full((_L,), jnp.sum(e * kept_v[r]))
            den = jnp.full((_L,), jnp.sum(e))
            acc = acc + jnp.where(lane0, num / den, 0.0)
    res_v[...] = acc
    pltpu.sync_copy(res_v, out_hbm.at[wid])


def _make_sc_topk(T):
    R = T // _NW
    mesh = plsc.VectorSubcoreMesh(
        core_axis_name="c", subcore_axis_name="s",
        num_cores=_NC, num_subcores=_NS)
    return pl.kernel(
        functools.partial(_topk_body, T=T, R=R),
        out_type=jax.ShapeDtypeStruct((_NW, _L), jnp.float32),
        mesh=mesh,
        scratch_types=[
            pltpu.VMEM((R, 2 * T), jnp.float32),
            pltpu.VMEM((_L,), jnp.float32),
            pltpu.SemaphoreType.DMA,
            pltpu.SemaphoreType.DMA,
        ],
        compiler_params=pltpu.CompilerParams(needs_layout_passes=False),
    )


def kernel(X, H, C, M, T):
    del T
    T, d = H.shape
    packed, mse = pl.pallas_call(
        functools.partial(_dense_kernel, T=T, d=d),
        out_shape=[
            jax.ShapeDtypeStruct((T, 2 * T), jnp.float32),
            jax.ShapeDtypeStruct((1, 1), jnp.float32),
        ],
    )(X, H, C, M)
    partials = _make_sc_topk(T)(packed)
    return mse[0, 0] + jnp.sum(partials)


# speedup vs baseline: 1.0023x; 1.0023x over previous
import functools

import jax
import jax.numpy as jnp
from jax import lax
from jax.experimental import pallas as pl
from jax.experimental.pallas import tpu as pltpu
from jax.experimental.pallas import tpu_sc as plsc

_L = 16
_NC, _NS = 2, 16
_NW = _NC * _NS


def _dense_kernel(x_ref, h_ref, c_ref, m_ref, packed_ref, mse_ref, *, T, d):
    H = h_ref[...]
    M = m_ref[...]
    dims = (((1,), (1,)), ((), ()))
    G = jax.lax.dot_general(H, H, dims, preferred_element_type=jnp.float32)
    GM = jax.lax.dot_general(M, M, dims, preferred_element_type=jnp.float32)
    nrm = jnp.sum(H * H, axis=1, keepdims=True)
    s = jnp.sum(H, axis=1, keepdims=True)
    mn = jnp.sum(M, axis=1, keepdims=True)

    sqd = nrm + nrm.T - 2.0 * G
    ds = s - s.T
    var = (sqd - ds * ds * (1.0 / d)) * (1.0 / (d - 1.0))
    good = var > 0.0
    score = jnp.where(good, jnp.sqrt(jnp.where(good, var, 1.0)), 0.0)

    msq = mn + mn.T - 2.0 * GM
    iota_r = jax.lax.broadcasted_iota(jnp.int32, (T, T), 0)
    iota_c = jax.lax.broadcasted_iota(jnp.int32, (T, T), 1)
    invalid = (iota_r == iota_c) | (msq <= 0.5)
    packed_ref[:, :T] = jnp.where(invalid, jnp.float32(9999.0), score)

    goodn = sqd > 0.0
    packed_ref[:, T:] = jnp.where(goodn, jnp.sqrt(jnp.where(goodn, sqd, 1.0)), 0.0)

    dd = x_ref[...] - H + c_ref[...]
    mse_ref[...] = jnp.reshape(jnp.sum(M * dd * dd), (1, 1))


def _merge16(ak, av, bk, bv):
    rk = lax.rev(bk, (0,))
    rv = lax.rev(bv, (0,))
    take_a = ak <= rk
    lo_k = jnp.where(take_a, ak, rk)
    lo_v = jnp.where(take_a, av, rv)
    return plsc.sort_key_val(lo_k, lo_v)


def _topk_body(packed_hbm, out_hbm, pk_v, res_v, *, T, R):
    wid = lax.axis_index("s") * _NC + lax.axis_index("c")
    base = wid * R
    pltpu.sync_copy(packed_hbm.at[pl.ds(base, R)], pk_v)
    lane = lax.iota(jnp.int32, _L)
    first8 = lane < 8
    lane0 = lane == 0
    acc = jnp.zeros((_L,), jnp.float32)
    zeros_idx = jnp.zeros((_L,), jnp.int32)
    kept_k = [jnp.full((_L,), 3.0e38, jnp.float32) for _ in range(R)]
    kept_v = [jnp.zeros((_L,), jnp.float32) for _ in range(R)]
    for c in range(T // _L):
        for r in range(R):
            ck = pk_v[r, pl.ds(c * _L, _L)]
            cv = pk_v[r, pl.ds(T + c * _L, _L)]
            sk, sv = plsc.sort_key_val(ck, cv)
            kept_k[r], kept_v[r] = _merge16(kept_k[r], kept_v[r], sk, sv)
    for r in range(R):
        v0 = kept_k[r][zeros_idx]
        e = jnp.where(first8, jnp.exp(v0 - kept_k[r]), 0.0)
        num = jnp.full((_L,), jnp.sum(e * kept_v[r]))
        den = jnp.full((_L,), jnp.sum(e))
        acc = acc + jnp.where(lane0, num / den, 0.0)
    res_v[...] = acc
    pltpu.sync_copy(res_v, out_hbm.at[wid])


def _make_sc_topk(T):
    R = T // _NW
    mesh = plsc.VectorSubcoreMesh(
        core_axis_name="c", subcore_axis_name="s",
        num_cores=_NC, num_subcores=_NS)
    return pl.kernel(
        functools.partial(_topk_body, T=T, R=R),
        out_type=jax.ShapeDtypeStruct((_NW, _L), jnp.float32),
        mesh=mesh,
        scratch_types=[
            pltpu.VMEM((R, 2 * T), jnp.float32),
            pltpu.VMEM((_L,), jnp.float32),
        ],
        compiler_params=pltpu.CompilerParams(needs_layout_passes=False),
    )


def kernel(X, H, C, M, T):
    del T
    T, d = H.shape
    packed, mse = pl.pallas_call(
        functools.partial(_dense_kernel, T=T, d=d),
        out_shape=[
            jax.ShapeDtypeStruct((T, 2 * T), jnp.float32),
            jax.ShapeDtypeStruct((1, 1), jnp.float32),
        ],
    )(X, H, C, M)
    partials = _make_sc_topk(T)(packed)
    return mse[0, 0] + jnp.sum(partials)


# speedup vs baseline: 1.0108x; 1.0084x over previous
import functools

import jax
import jax.numpy as jnp
from jax import lax
from jax.experimental import pallas as pl
from jax.experimental.pallas import tpu as pltpu
from jax.experimental.pallas import tpu_sc as plsc

_L = 16
_NC, _NS = 2, 16
_NW = _NC * _NS


def _dense_kernel(x_ref, h_ref, c_ref, m_ref, packed_ref, mse_ref, *, T, d):
    H = h_ref[...]
    M = m_ref[...]
    dims = (((1,), (1,)), ((), ()))
    G = jax.lax.dot_general(H, H, dims, preferred_element_type=jnp.float32)
    GM = jax.lax.dot_general(M, M, dims, preferred_element_type=jnp.float32)
    nrm = jnp.sum(H * H, axis=1, keepdims=True)
    s = jnp.sum(H, axis=1, keepdims=True)
    mn = jnp.sum(M, axis=1, keepdims=True)

    sqd = nrm + nrm.T - 2.0 * G
    ds = s - s.T
    var = (sqd - ds * ds * (1.0 / d)) * (1.0 / (d - 1.0))
    good = var > 0.0
    score = jnp.where(good, jnp.sqrt(jnp.where(good, var, 1.0)), 0.0)

    msq = mn + mn.T - 2.0 * GM
    iota_r = jax.lax.broadcasted_iota(jnp.int32, (T, T), 0)
    iota_c = jax.lax.broadcasted_iota(jnp.int32, (T, T), 1)
    invalid = (iota_r == iota_c) | (msq <= 0.5)
    packed_ref[:, :T] = jnp.where(invalid, jnp.float32(9999.0), score)

    goodn = sqd > 0.0
    packed_ref[:, T:] = jnp.where(goodn, jnp.sqrt(jnp.where(goodn, sqd, 1.0)), 0.0)

    dd = x_ref[...] - H + c_ref[...]
    mse_ref[...] = jnp.reshape(jnp.sum(M * dd * dd), (1, 1))


def _merge16(ak, av, dk, dv):
    take_a = ak <= dk
    lo_k = jnp.where(take_a, ak, dk)
    lo_v = jnp.where(take_a, av, dv)
    return plsc.sort_key_val(lo_k, lo_v)


def _topk_body(packed_hbm, out_hbm, pk_v, res_v, *, T, R):
    wid = lax.axis_index("s") * _NC + lax.axis_index("c")
    base = wid * R
    pltpu.sync_copy(packed_hbm.at[pl.ds(base, R)], pk_v)
    lane = lax.iota(jnp.int32, _L)
    first8 = lane < 8
    lane0 = lane == 0
    acc = jnp.zeros((_L,), jnp.float32)
    zeros_idx = jnp.zeros((_L,), jnp.int32)
    kept_k = [jnp.full((_L,), 3.0e38, jnp.float32) for _ in range(R)]
    kept_v = [jnp.zeros((_L,), jnp.float32) for _ in range(R)]
    for c in range(T // _L):
        for r in range(R):
            ck = pk_v[r, pl.ds(c * _L, _L)]
            cv = pk_v[r, pl.ds(T + c * _L, _L)]
            sk, sv = plsc.sort_key_val(ck, cv, descending=True)
            kept_k[r], kept_v[r] = _merge16(kept_k[r], kept_v[r], sk, sv)
    for r in range(R):
        v0 = kept_k[r][zeros_idx]
        e = jnp.where(first8, jnp.exp(v0 - kept_k[r]), 0.0)
        num = jnp.full((_L,), jnp.sum(e * kept_v[r]))
        den = jnp.full((_L,), jnp.sum(e))
        acc = acc + jnp.where(lane0, num / den, 0.0)
    res_v[...] = acc
    pltpu.sync_copy(res_v, out_hbm.at[wid])


def _make_sc_topk(T):
    R = T // _NW
    mesh = plsc.VectorSubcoreMesh(
        core_axis_name="c", subcore_axis_name="s",
        num_cores=_NC, num_subcores=_NS)
    return pl.kernel(
        functools.partial(_topk_body, T=T, R=R),
        out_type=jax.ShapeDtypeStruct((_NW, _L), jnp.float32),
        mesh=mesh,
        scratch_types=[
            pltpu.VMEM((R, 2 * T), jnp.float32),
            pltpu.VMEM((_L,), jnp.float32),
        ],
        compiler_params=pltpu.CompilerParams(needs_layout_passes=False),
    )


def kernel(X, H, C, M, T):
    del T
    T, d = H.shape
    packed, mse = pl.pallas_call(
        functools.partial(_dense_kernel, T=T, d=d),
        out_shape=[
            jax.ShapeDtypeStruct((T, 2 * T), jnp.float32),
            jax.ShapeDtypeStruct((1, 1), jnp.float32),
        ],
    )(X, H, C, M)
    partials = _make_sc_topk(T)(packed)
    return mse[0, 0] + jnp.sum(partials)
